# baseline (speedup 1.0000x reference)
import jax
import jax.numpy as jnp
from jax.experimental import pallas as pl
from jax.experimental.pallas import tpu as pltpu

_P = 1024
_D = 4096
_C = 128
_NCHUNK = _D // _C
_K = 24


def _finite_or_zero(x):
    return jnp.where(jnp.isfinite(x), x, 0.0)


def _match_block(sig_ref, dbt_ref, pay_ref, bias_ref, dcol_ref, drow_ref,
                 t2o_ref, b1o_ref, mdo_ref, dba_ref):
    @pl.when(pl.program_id(0) == 0)
    def _prep():
        dbt = dbt_ref[...]
        se_col = (dcol_ref[:, 0:1] * 0.001 < 0.001).astype(jnp.float32)
        dbm = dbt * se_col
        dn2 = jnp.sum(dbm * dbm, axis=0, keepdims=True)
        dbn = _finite_or_zero(dbm / jnp.sqrt(dn2))
        dbsq_half = 0.5 * jnp.sum(dbn * dbn, axis=0, keepdims=True)
        dba_ref[0:16, :] = -dbn
        dba_ref[16:17, :] = dbsq_half + bias_ref[0:1, :]
        dba_ref[17:_K, :] = jnp.zeros((_K - 17, _D), jnp.float32)

    sig = sig_ref[...]
    sn2 = jnp.sum(sig * sig, axis=1, keepdims=True)
    sign = _finite_or_zero(sig / jnp.sqrt(sn2))
    pix = (sign[:, 0:1] > 1e-6).astype(jnp.float32)
    se_row = (drow_ref[0:1, :] * 0.001 < 0.001).astype(jnp.float32)
    q = sign * se_row
    qn2 = jnp.sum(q * q, axis=1, keepdims=True)
    qn = _finite_or_zero(q / jnp.sqrt(qn2))
    qa = jnp.concatenate(
        [qn, jnp.ones((_P, 1), jnp.float32),
         jnp.zeros((_P, _K - 17), jnp.float32)], axis=1)

    keys = jax.lax.dot_general(
        qa, dba_ref[...], (((1,), (0,)), ((), ())),
        preferred_element_type=jnp.float32)

    pay = pay_ref[...]
    run_key = keys[:, 0:_C]
    run_pay = jnp.broadcast_to(pay[0:1, :], run_key.shape)
    for c in range(1, _NCHUNK):
        kc = keys[:, c * _C:(c + 1) * _C]
        cond = kc < run_key
        run_key = jnp.where(cond, kc, run_key)
        run_pay = jnp.where(cond, pay[c:c + 1, :], run_pay)

    m = jnp.min(run_key, axis=1, keepdims=True)
    mind = jnp.sqrt(jnp.maximum(1.0 + 2.0 * m, 1e-12))
    lane = jax.lax.broadcasted_iota(jnp.int32, run_key.shape, 1)
    first = jnp.min(jnp.where(run_key == m, lane, _C), axis=1, keepdims=True)
    sel = lane == first
    packed = jnp.sum(jnp.where(sel, run_pay, 0), axis=1, keepdims=True)

    t2v = jax.lax.bitcast_convert_type(
        jnp.bitwise_and(packed, jnp.int32(-65536)), jnp.float32)
    b1v = jax.lax.bitcast_convert_type(
        jnp.left_shift(packed, 16), jnp.float32)

    t2o_ref[...] = t2v * pix
    b1o_ref[...] = b1v * pix
    mdo_ref[...] = mind * pix


def kernel(slice_signal, db_torch_mag, db_t2s_s, db_b1s, delta_t_r2p_ms):
    n_db, etl = db_torch_mag.shape
    sig = slice_signal.reshape(-1, etl)
    npix = sig.shape[0]
    pad = _D - n_db

    dbt = jnp.pad(db_torch_mag, ((0, pad), (0, 0))).T
    t2u = jax.lax.bitcast_convert_type(
        db_t2s_s.astype(jnp.bfloat16), jnp.uint16).astype(jnp.uint32)
    b1u = jax.lax.bitcast_convert_type(
        db_b1s.astype(jnp.bfloat16), jnp.uint16).astype(jnp.uint32)
    pay = ((t2u << 16) | b1u).astype(jnp.int32)
    pay = jnp.pad(pay, (0, pad)).reshape(_NCHUNK, _C)
    bias = jnp.pad(jnp.zeros((n_db,), jnp.float32), (0, pad),
                   constant_values=1e30)
    bias = jnp.broadcast_to(bias[None, :], (8, _D))
    dcol = jnp.broadcast_to(delta_t_r2p_ms[:, None], (etl, 8))
    drow = jnp.broadcast_to(delta_t_r2p_ms[None, :], (8, etl))

    grid = (npix // _P,)
    t2o, b1o, mdo = pl.pallas_call(
        _match_block,
        grid=grid,
        in_specs=[
            pl.BlockSpec((_P, etl), lambda i: (i, 0)),
            pl.BlockSpec((etl, _D), lambda i: (0, 0)),
            pl.BlockSpec((_NCHUNK, _C), lambda i: (0, 0)),
            pl.BlockSpec((8, _D), lambda i: (0, 0)),
            pl.BlockSpec((etl, 8), lambda i: (0, 0)),
            pl.BlockSpec((8, etl), lambda i: (0, 0)),
        ],
        out_specs=[
            pl.BlockSpec((_P, 1), lambda i: (i, 0)),
            pl.BlockSpec((_P, 1), lambda i: (i, 0)),
            pl.BlockSpec((_P, 1), lambda i: (i, 0)),
        ],
        out_shape=[
            jax.ShapeDtypeStruct((npix, 1), jnp.float32),
            jax.ShapeDtypeStruct((npix, 1), jnp.float32),
            jax.ShapeDtypeStruct((npix, 1), jnp.float32),
        ],
        scratch_shapes=[pltpu.VMEM((_K, _D), jnp.float32)],
    )(sig, dbt, pay, bias, dcol, drow)

    return jnp.concatenate([t2o, b1o, mdo], axis=1)


# speedup vs baseline: 1.0198x; 1.0198x over previous
import jax
import jax.numpy as jnp
from jax.experimental import pallas as pl
from jax.experimental.pallas import tpu as pltpu

_P = 2048
_D = 4096
_C = 128
_NCHUNK = _D // _C
_K = 24


def _finite_or_zero(x):
    return jnp.where(jnp.isfinite(x), x, 0.0)


def _match_block(sig_ref, dbt_ref, pay_ref, bias_ref, dcol_ref, drow_ref,
                 t2o_ref, b1o_ref, mdo_ref, dba_ref):
    @pl.when(pl.program_id(0) == 0)
    def _prep():
        dbt = dbt_ref[...]
        se_col = (dcol_ref[:, 0:1] * 0.001 < 0.001).astype(jnp.float32)
        dbm = dbt * se_col
        dn2 = jnp.sum(dbm * dbm, axis=0, keepdims=True)
        dbn = _finite_or_zero(dbm / jnp.sqrt(dn2))
        dbsq_half = 0.5 * jnp.sum(dbn * dbn, axis=0, keepdims=True)
        dba_ref[0:16, :] = -dbn
        dba_ref[16:17, :] = dbsq_half + bias_ref[0:1, :]
        dba_ref[17:_K, :] = jnp.zeros((_K - 17, _D), jnp.float32)

    sig = sig_ref[...]
    sn2 = jnp.sum(sig * sig, axis=1, keepdims=True)
    sign = _finite_or_zero(sig / jnp.sqrt(sn2))
    pix = (sign[:, 0:1] > 1e-6).astype(jnp.float32)
    se_row = (drow_ref[0:1, :] * 0.001 < 0.001).astype(jnp.float32)
    q = sign * se_row
    qn2 = jnp.sum(q * q, axis=1, keepdims=True)
    qn = _finite_or_zero(q / jnp.sqrt(qn2))
    qa = jnp.concatenate(
        [qn, jnp.ones((_P, 1), jnp.float32),
         jnp.zeros((_P, _K - 17), jnp.float32)], axis=1)

    keys = jax.lax.dot_general(
        qa, dba_ref[...], (((1,), (0,)), ((), ())),
        preferred_element_type=jnp.float32)

    pay = pay_ref[...]
    run_key = keys[:, 0:_C]
    run_pay = jnp.broadcast_to(pay[0:1, :], run_key.shape)
    for c in range(1, _NCHUNK):
        kc = keys[:, c * _C:(c + 1) * _C]
        cond = kc < run_key
        run_key = jnp.where(cond, kc, run_key)
        run_pay = jnp.where(cond, pay[c:c + 1, :], run_pay)

    m = jnp.min(run_key, axis=1, keepdims=True)
    mind = jnp.sqrt(jnp.maximum(1.0 + 2.0 * m, 1e-12))
    lane = jax.lax.broadcasted_iota(jnp.int32, run_key.shape, 1)
    first = jnp.min(jnp.where(run_key == m, lane, _C), axis=1, keepdims=True)
    sel = lane == first
    packed = jnp.sum(jnp.where(sel, run_pay, 0), axis=1, keepdims=True)

    t2v = jax.lax.bitcast_convert_type(
        jnp.bitwise_and(packed, jnp.int32(-65536)), jnp.float32)
    b1v = jax.lax.bitcast_convert_type(
        jnp.left_shift(packed, 16), jnp.float32)

    t2o_ref[...] = t2v * pix
    b1o_ref[...] = b1v * pix
    mdo_ref[...] = mind * pix


def kernel(slice_signal, db_torch_mag, db_t2s_s, db_b1s, delta_t_r2p_ms):
    n_db, etl = db_torch_mag.shape
    sig = slice_signal.reshape(-1, etl)
    npix = sig.shape[0]
    pad = _D - n_db

    dbt = jnp.pad(db_torch_mag, ((0, pad), (0, 0))).T
    t2u = jax.lax.bitcast_convert_type(
        db_t2s_s.astype(jnp.bfloat16), jnp.uint16).astype(jnp.uint32)
    b1u = jax.lax.bitcast_convert_type(
        db_b1s.astype(jnp.bfloat16), jnp.uint16).astype(jnp.uint32)
    pay = ((t2u << 16) | b1u).astype(jnp.int32)
    pay = jnp.pad(pay, (0, pad)).reshape(_NCHUNK, _C)
    bias = jnp.pad(jnp.zeros((n_db,), jnp.float32), (0, pad),
                   constant_values=1e30)
    bias = jnp.broadcast_to(bias[None, :], (8, _D))
    dcol = jnp.broadcast_to(delta_t_r2p_ms[:, None], (etl, 8))
    drow = jnp.broadcast_to(delta_t_r2p_ms[None, :], (8, etl))

    grid = (npix // _P,)
    t2o, b1o, mdo = pl.pallas_call(
        _match_block,
        grid=grid,
        in_specs=[
            pl.BlockSpec((_P, etl), lambda i: (i, 0)),
            pl.BlockSpec((etl, _D), lambda i: (0, 0)),
            pl.BlockSpec((_NCHUNK, _C), lambda i: (0, 0)),
            pl.BlockSpec((8, _D), lambda i: (0, 0)),
            pl.BlockSpec((etl, 8), lambda i: (0, 0)),
            pl.BlockSpec((8, etl), lambda i: (0, 0)),
        ],
        out_specs=[
            pl.BlockSpec((_P, 1), lambda i: (i, 0)),
            pl.BlockSpec((_P, 1), lambda i: (i, 0)),
            pl.BlockSpec((_P, 1), lambda i: (i, 0)),
        ],
        out_shape=[
            jax.ShapeDtypeStruct((npix, 1), jnp.float32),
            jax.ShapeDtypeStruct((npix, 1), jnp.float32),
            jax.ShapeDtypeStruct((npix, 1), jnp.float32),
        ],
        scratch_shapes=[pltpu.VMEM((_K, _D), jnp.float32)],
    )(sig, dbt, pay, bias, dcol, drow)

    return jnp.concatenate([t2o, b1o, mdo], axis=1)


# speedup vs baseline: 1.0886x; 1.0674x over previous
import functools

import jax
import jax.numpy as jnp
from jax import lax
from jax.experimental import pallas as pl
from jax.experimental.pallas import tpu as pltpu
from jax.experimental.pallas import tpu_sc as plsc

_P = 2048
_D = 4096
_C = 128
_NCHUNK = _D // _C
_K = 24


def _finite_or_zero(x):
    return jnp.where(jnp.isfinite(x), x, 0.0)


def _match_block(sig_ref, dbt_ref, pay_ref, bias_ref, dcol_ref, drow_ref,
                 idxo_ref, mdo_ref, dba_ref):
    @pl.when(pl.program_id(0) == 0)
    def _prep():
        dbt = dbt_ref[...]
        se_col = (dcol_ref[:, 0:1] * 0.001 < 0.001).astype(jnp.float32)
        dbm = dbt * se_col
        dn2 = jnp.sum(dbm * dbm, axis=0, keepdims=True)
        dbn = _finite_or_zero(dbm / jnp.sqrt(dn2))
        dbsq_half = 0.5 * jnp.sum(dbn * dbn, axis=0, keepdims=True)
        dba_ref[0:16, :] = -dbn
        dba_ref[16:17, :] = dbsq_half + bias_ref[0:1, :]
        dba_ref[17:_K, :] = jnp.zeros((_K - 17, _D), jnp.float32)

    sig = sig_ref[...]
    sn2 = jnp.sum(sig * sig, axis=1, keepdims=True)
    sign = _finite_or_zero(sig / jnp.sqrt(sn2))
    pix = sign[:, 0:1] > 1e-6
    se_row = (drow_ref[0:1, :] * 0.001 < 0.001).astype(jnp.float32)
    q = sign * se_row
    qn2 = jnp.sum(q * q, axis=1, keepdims=True)
    qn = _finite_or_zero(q / jnp.sqrt(qn2))
    qa = jnp.concatenate(
        [qn, jnp.ones((_P, 1), jnp.float32),
         jnp.zeros((_P, _K - 17), jnp.float32)], axis=1)

    keys = jax.lax.dot_general(
        qa, dba_ref[...], (((1,), (0,)), ((), ())),
        preferred_element_type=jnp.float32)

    pay = pay_ref[...]
    run_key = keys[:, 0:_C]
    run_idx = jnp.broadcast_to(pay[0:1, :], run_key.shape)
    for c in range(1, _NCHUNK):
        kc = keys[:, c * _C:(c + 1) * _C]
        cond = kc < run_key
        run_key = jnp.where(cond, kc, run_key)
        run_idx = jnp.where(cond, pay[c:c + 1, :], run_idx)

    m = jnp.min(run_key, axis=1, keepdims=True)
    mind = jnp.sqrt(jnp.maximum(1.0 + 2.0 * m, 1e-12))
    idxv = jnp.min(jnp.where(run_key == m, run_idx, _D), axis=1,
                   keepdims=True)
    idxo_ref[...] = jnp.where(pix, idxv, _D - 1)
    mdo_ref[...] = jnp.where(pix, mind, 0.0)


def _sc_gather(idx_hbm, t2_hbm, b1_hbm, t2o_hbm, b1o_hbm,
               t2v, b1v, idxv, ot2, ob1):
    info = plsc.get_sparse_core_info()
    nw = info.num_cores * info.num_subcores
    wid = lax.axis_index("s") * info.num_cores + lax.axis_index("c")
    bw = idx_hbm.shape[0] // nw
    base = wid * bw
    pltpu.sync_copy(t2_hbm, t2v)
    pltpu.sync_copy(b1_hbm, b1v)
    pltpu.sync_copy(idx_hbm.at[pl.ds(base, bw)], idxv)

    def body(i, carry):
        off = i * 16
        iv = idxv[pl.ds(off, 16)]
        ot2[pl.ds(off, 16)] = plsc.load_gather(t2v, [iv])
        ob1[pl.ds(off, 16)] = plsc.load_gather(b1v, [iv])
        return carry

    lax.fori_loop(0, bw // 16, body, 0)
    pltpu.sync_copy(ot2, t2o_hbm.at[pl.ds(base, bw)])
    pltpu.sync_copy(ob1, b1o_hbm.at[pl.ds(base, bw)])


def kernel(slice_signal, db_torch_mag, db_t2s_s, db_b1s, delta_t_r2p_ms):
    n_db, etl = db_torch_mag.shape
    sig = slice_signal.reshape(-1, etl)
    npix = sig.shape[0]
    pad = _D - n_db

    dbt = jnp.pad(db_torch_mag, ((0, pad), (0, 0))).T
    pay = jnp.arange(_D, dtype=jnp.int32).reshape(_NCHUNK, _C)
    bias = jnp.pad(jnp.zeros((n_db,), jnp.float32), (0, pad),
                   constant_values=1e30)
    bias = jnp.broadcast_to(bias[None, :], (8, _D))
    dcol = jnp.broadcast_to(delta_t_r2p_ms[:, None], (etl, 8))
    drow = jnp.broadcast_to(delta_t_r2p_ms[None, :], (8, etl))

    grid = (npix // _P,)
    idxo, mdo = pl.pallas_call(
        _match_block,
        grid=grid,
        in_specs=[
            pl.BlockSpec((_P, etl), lambda i: (i, 0)),
            pl.BlockSpec((etl, _D), lambda i: (0, 0)),
            pl.BlockSpec((_NCHUNK, _C), lambda i: (0, 0)),
            pl.BlockSpec((8, _D), lambda i: (0, 0)),
            pl.BlockSpec((etl, 8), lambda i: (0, 0)),
            pl.BlockSpec((8, etl), lambda i: (0, 0)),
        ],
        out_specs=[
            pl.BlockSpec((_P, 1), lambda i: (i, 0)),
            pl.BlockSpec((_P, 1), lambda i: (i, 0)),
        ],
        out_shape=[
            jax.ShapeDtypeStruct((npix, 1), jnp.int32),
            jax.ShapeDtypeStruct((npix, 1), jnp.float32),
        ],
        scratch_shapes=[pltpu.VMEM((_K, _D), jnp.float32)],
    )(sig, dbt, pay, bias, dcol, drow)

    t2tab = jnp.pad(db_t2s_s, (0, pad))
    b1tab = jnp.pad(db_b1s, (0, pad))
    info = plsc.get_sparse_core_info()
    nw = info.num_cores * info.num_subcores
    bw = npix // nw
    sc = pl.kernel(
        _sc_gather,
        mesh=plsc.VectorSubcoreMesh(core_axis_name="c", subcore_axis_name="s"),
        out_type=[
            jax.ShapeDtypeStruct((npix,), jnp.float32),
            jax.ShapeDtypeStruct((npix,), jnp.float32),
        ],
        scratch_types=[
            pltpu.VMEM((_D,), jnp.float32),
            pltpu.VMEM((_D,), jnp.float32),
            pltpu.VMEM((bw,), jnp.int32),
            pltpu.VMEM((bw,), jnp.float32),
            pltpu.VMEM((bw,), jnp.float32),
        ],
        compiler_params=pltpu.CompilerParams(needs_layout_passes=False),
    )
    t2o, b1o = sc(idxo.reshape(npix), t2tab, b1tab)

    return jnp.concatenate([t2o[:, None], b1o[:, None], mdo], axis=1)
